# baseline (speedup 1.0000x reference)
import functools
import numpy as np
import jax
import jax.numpy as jnp
from jax import lax
from jax.experimental import pallas as pl
from jax.experimental.pallas import tpu as pltpu
from jax.experimental.pallas import tpu_sc as plsc

D_MODEL = 1024
D_FF = 4096
D_NEURON = 128
N_HEADS = 8
HD = D_NEURON // N_HEADS
K = 8

def _gelu(v):
    return v * (lax.erf(v / np.sqrt(2.0)) + 1.0) / 2.0


def _ln_r1_body(x_ref, w1t_ref, g_ref, b_ref, h_ref):
    x = x_ref[...]
    m = jnp.mean(x, axis=1, keepdims=True)
    xc = x - m
    v = jnp.mean(xc * xc, axis=1, keepdims=True)
    xn = xc / jnp.sqrt(v + 1e-5) * g_ref[...] + b_ref[...]
    h_ref[...] = _gelu(jnp.dot(xn, w1t_ref[...],
                               preferred_element_type=jnp.float32))


def _ln_router1(xf, w1t, g, b, bm=512, interpret=False):
    T, D = xf.shape
    return pl.pallas_call(
        _ln_r1_body,
        grid=(T // bm,),
        in_specs=[
            pl.BlockSpec((bm, D), lambda i: (i, 0)),
            pl.BlockSpec((D, D), lambda i: (0, 0)),
            pl.BlockSpec((1, D), lambda i: (0, 0)),
            pl.BlockSpec((1, D), lambda i: (0, 0)),
        ],
        out_specs=pl.BlockSpec((bm, D), lambda i: (i, 0)),
        out_shape=jax.ShapeDtypeStruct((T, D), jnp.float32),
        interpret=interpret,
    )(xf, w1t, g, b)


def _mm_body(a_ref, b_ref, o_ref):
    o_ref[...] = jnp.dot(a_ref[...], b_ref[...],
                         preferred_element_type=jnp.float32)


def _mm(a, b, bm=512, bn=2048, interpret=False):
    M, Kd = a.shape
    N = b.shape[1]
    return pl.pallas_call(
        _mm_body,
        grid=(N // bn, M // bm),
        in_specs=[
            pl.BlockSpec((bm, Kd), lambda j, i: (i, 0)),
            pl.BlockSpec((Kd, bn), lambda j, i: (0, j)),
        ],
        out_specs=pl.BlockSpec((bm, bn), lambda j, i: (i, j)),
        out_shape=jax.ShapeDtypeStruct((M, N), jnp.float32),
        interpret=interpret,
    )(a, b)


def _topk_body(s_ref, xp_ref, idx_ref, base_ref):
    s = s_ref[...]
    xp = xp_ref[...]
    iota = lax.broadcasted_iota(jnp.int32, s.shape, 1)
    big = jnp.int32(2 ** 30)
    for j in range(K):
        m = jnp.max(s, axis=1, keepdims=True)
        eq = s == m
        ij = jnp.min(jnp.where(eq, iota, big), axis=1)
        sel = iota == ij[:, None]
        bj = jnp.sum(jnp.where(sel, xp, jnp.float32(0.0)), axis=1)
        s = jnp.where(sel, jnp.float32(-jnp.inf), s)
        idx_ref[j, :] = ij
        base_ref[j, :] = _gelu(bj)


def _topk(scores, xpat, bm=512, interpret=False):
    T, F = scores.shape
    return pl.pallas_call(
        _topk_body,
        grid=(T // bm,),
        in_specs=[
            pl.BlockSpec((bm, F), lambda i: (i, 0)),
            pl.BlockSpec((bm, F), lambda i: (i, 0)),
        ],
        out_specs=[
            pl.BlockSpec((K, bm), lambda i: (0, i)),
            pl.BlockSpec((K, bm), lambda i: (0, i)),
        ],
        out_shape=[
            jax.ShapeDtypeStruct((K, T), jnp.int32),
            jax.ShapeDtypeStruct((K, T), jnp.float32),
        ],
        interpret=interpret,
    )(scores, xpat)


_NW = 32
_VCH = 128
_DCH = 32
_NB = 2


@functools.lru_cache(maxsize=4)
def _make_sc_gather(b8):
    bpw = b8 // _NW

    @functools.partial(
        pl.kernel,
        out_type=[
            jax.ShapeDtypeStruct((b8, D_NEURON), jnp.float32),
            jax.ShapeDtypeStruct((b8, D_MODEL), jnp.float32),
        ],
        mesh=plsc.VectorSubcoreMesh(core_axis_name="c", subcore_axis_name="s"),
        scratch_types=[
            pltpu.VMEM((bpw,), jnp.int32),
            pltpu.VMEM((_VCH, D_NEURON), jnp.float32),
            pltpu.VMEM((_VCH, D_NEURON), jnp.float32),
        ] + [pltpu.VMEM((_DCH, D_MODEL), jnp.float32)] * _NB + [
            pltpu.SemaphoreType.DMA,
            pltpu.SemaphoreType.DMA,
        ] + [pltpu.SemaphoreType.DMA] * (2 * _NB),
    )
    def _sc_gather(vecs_hbm, dirs_hbm, idx_hbm, sems_out, dirsg_out,
                   idx_v, vbufA, vbufB, *drest):
        dbufs = list(drest[:_NB])
        gA, gB = drest[_NB], drest[_NB + 1]
        dg = list(drest[_NB + 2:2 * _NB + 2])
        ds = list(drest[2 * _NB + 2:])
        wid = lax.axis_index("s") * 2 + lax.axis_index("c")
        base = wid * bpw
        pltpu.sync_copy(idx_hbm.at[pl.ds(base, bpw)], idx_v)

        def ring(table, out_hbm, bufA, bufB, ch, nch):
            def gsrc(i):
                return table.at[idx_v.at[pl.ds(i * ch, ch)]]

            def orows(i):
                return out_hbm.at[pl.ds(base + i * ch, ch)]

            pltpu.async_copy(gsrc(0), bufA, gA)
            pltpu.async_copy(gsrc(1), bufB, gB)

            def body(p, c):
                e = 2 * p
                pltpu.make_async_copy(gsrc(e), bufA, gA).wait()
                pltpu.async_copy(bufA, orows(e), ds[0])
                pltpu.make_async_copy(bufA, orows(e), ds[0]).wait()
                pltpu.async_copy(gsrc(e + 2), bufA, gA)
                pltpu.make_async_copy(gsrc(e + 1), bufB, gB).wait()
                pltpu.async_copy(bufB, orows(e + 1), ds[1])
                pltpu.make_async_copy(bufB, orows(e + 1), ds[1]).wait()
                pltpu.async_copy(gsrc(e + 3), bufB, gB)
                return c

            lax.fori_loop(0, nch // 2 - 1, body, 0)
            e = nch - 2
            pltpu.make_async_copy(gsrc(e), bufA, gA).wait()
            pltpu.async_copy(bufA, orows(e), ds[0])
            pltpu.make_async_copy(gsrc(e + 1), bufB, gB).wait()
            pltpu.async_copy(bufB, orows(e + 1), ds[1])
            pltpu.make_async_copy(bufA, orows(e), ds[0]).wait()
            pltpu.make_async_copy(bufB, orows(e + 1), ds[1]).wait()

        ring(vecs_hbm, sems_out, vbufA, vbufB, _VCH, bpw // _VCH)
        ring(dirs_hbm, dirsg_out, dbufs[0], dbufs[1], _DCH, bpw // _DCH)

    return _sc_gather


def _df_body(sems_ref, base_ref, dirs_ref, inw_ref, inb_ref, outw_ref,
             outb_ref, w1_ref, b1_ref, w2_ref, b2_ref, rbd_ref, ebd_ref,
             o_ref):
    G = o_ref.shape[0]
    s = sems_ref[...].reshape(K * G, D_NEURON)
    qkv = jnp.dot(s, inw_ref[...], preferred_element_type=jnp.float32) + inb_ref[...]
    q = qkv[:, :D_NEURON]
    k = qkv[:, D_NEURON:2 * D_NEURON]
    v = qkv[:, 2 * D_NEURON:]
    rbd = rbd_ref[...]
    ebd = ebd_ref[...]
    ls = []
    for j in range(K):
        kj = k[j * G:(j + 1) * G, :]
        prod = jnp.concatenate(
            [q[i * G:(i + 1) * G, :] * kj for i in range(K)], axis=1)
        ls.append(jnp.dot(prod, rbd, preferred_element_type=jnp.float32))
    m = ls[0]
    for j in range(1, K):
        m = jnp.maximum(m, ls[j])
    es = [jnp.exp(l - m) for l in ls]
    ssum = es[0]
    for j in range(1, K):
        ssum = ssum + es[j]
    acc = None
    for j in range(K):
        pexp = jnp.dot(es[j] / ssum, ebd, preferred_element_type=jnp.float32)
        vj = v[j * G:(j + 1) * G, :]
        vt = jnp.concatenate([vj] * K, axis=1)
        t = pexp * vt
        acc = t if acc is None else acc + t
    att = jnp.dot(acc, outw_ref[...], preferred_element_type=jnp.float32) + outb_ref[...]
    hmid = _gelu(jnp.dot(att, w1_ref[...], preferred_element_type=jnp.float32) + b1_ref[...])
    wlog = jnp.dot(hmid, w2_ref[...], preferred_element_type=jnp.float32) + b2_ref[...]
    w = 1.0 / (1.0 + jnp.exp(-wlog))
    coef = base_ref[0].transpose() * w
    o = None
    for i in range(K):
        t = coef[:, i:i + 1] * dirs_ref[i]
        o = t if o is None else o + t
    o_ref[...] = o


def _df(sems3, baseT, dirs3, inw, inb, outw, outb, w1, b1, w2,
        b2, rbd, ebd, gtok=128, interpret=False):
    T = baseT.shape[1]
    full = lambda shape: pl.BlockSpec(shape, lambda i: tuple(0 for _ in shape))
    return pl.pallas_call(
        _df_body,
        grid=(T // gtok,),
        in_specs=[
            pl.BlockSpec((K, gtok, D_NEURON), lambda i: (0, i, 0)),
            pl.BlockSpec((1, K, gtok), lambda i: (0, 0, i)),
            pl.BlockSpec((K, gtok, D_MODEL), lambda i: (0, i, 0)),
            full((D_NEURON, 3 * D_NEURON)),
            full((1, 3 * D_NEURON)),
            full((K * D_NEURON, K * D_NEURON)),
            full((1, K * D_NEURON)),
            full((K * D_NEURON, K * D_NEURON // 2)),
            full((1, K * D_NEURON // 2)),
            full((K * D_NEURON // 2, K)),
            full((1, 1)),
            full((K * D_NEURON, K * N_HEADS)),
            full((K * N_HEADS, K * D_NEURON)),
        ],
        out_specs=pl.BlockSpec((gtok, D_MODEL), lambda i: (i, 0)),
        out_shape=jax.ShapeDtypeStruct((T, D_MODEL), jnp.float32),
        interpret=interpret,
    )(sems3, baseT.reshape(1, K, T), dirs3, inw, inb, outw, outb, w1, b1,
      w2, b2, rbd, ebd)


def kernel(x, top_k, neuron_patterns, neuron_vecs, neuron_out_dirs,
           W_router_1, W_router_2, ln_gamma, ln_beta, in_proj_w, in_proj_b,
           out_proj_w, out_proj_b, wp_w1, wp_b1, wp_w2, wp_b2):
    B, S, D = x.shape
    xf = x.reshape(-1, D)

    h = _ln_router1(xf, W_router_1.T, ln_gamma.reshape(1, -1),
                    ln_beta.reshape(1, -1))
    scores = _mm(h, W_router_2.T)
    xpat = _mm(xf, neuron_patterns.T)

    hmask = (np.arange(D_NEURON)[:, None] // HD ==
             np.arange(N_HEADS)[None, :]).astype(np.float32)
    rbd_np = np.kron(np.eye(K, dtype=np.float32),
                     hmask * np.float32(1.0 / np.sqrt(float(HD))))
    ebd_np = np.kron(np.eye(K, dtype=np.float32), hmask.T)
    rbd = jnp.asarray(rbd_np)
    ebd = jnp.asarray(ebd_np)

    eye8 = jnp.eye(K, dtype=jnp.float32)
    owbd = jnp.kron(eye8, out_proj_w.T)
    outb_t = jnp.tile(out_proj_b, K).reshape(1, -1)
    w1bd = jnp.kron(eye8, wp_w1.T)
    b1_t = jnp.tile(wp_b1, K).reshape(1, -1)
    w2bd = jnp.kron(eye8, wp_w2.reshape(-1, 1))

    T = xf.shape[0]
    idxT, baseT = _topk(scores, xpat)
    idx_flat = idxT.reshape(-1)
    sems_flat, dirs_flat = _make_sc_gather(T * K)(neuron_vecs,
                                                  neuron_out_dirs, idx_flat)
    sems3 = sems_flat.reshape(K, T, D_NEURON)
    dirs3 = dirs_flat.reshape(K, T, D_MODEL)
    out = _df(sems3, baseT, dirs3,
              in_proj_w.T, in_proj_b.reshape(1, -1),
              owbd, outb_t, w1bd, b1_t, w2bd,
              wp_b2.reshape(1, 1), rbd, ebd)
    return out.reshape(B, S, D)


# speedup vs baseline: 1.0327x; 1.0327x over previous
import functools
import numpy as np
import jax
import jax.numpy as jnp
from jax import lax
from jax.experimental import pallas as pl
from jax.experimental.pallas import tpu as pltpu
from jax.experimental.pallas import tpu_sc as plsc

D_MODEL = 1024
D_FF = 4096
D_NEURON = 128
N_HEADS = 8
HD = D_NEURON // N_HEADS
K = 8

def _gelu(v):
    return v * (lax.erf(v / np.sqrt(2.0)) + 1.0) / 2.0


def _ln_r1_body(x_ref, w1t_ref, g_ref, b_ref, h_ref):
    x = x_ref[...]
    m = jnp.mean(x, axis=1, keepdims=True)
    xc = x - m
    v = jnp.mean(xc * xc, axis=1, keepdims=True)
    xn = xc / jnp.sqrt(v + 1e-5) * g_ref[...] + b_ref[...]
    h_ref[...] = _gelu(jnp.dot(xn, w1t_ref[...],
                               preferred_element_type=jnp.float32))


def _ln_router1(xf, w1t, g, b, bm=512, interpret=False):
    T, D = xf.shape
    return pl.pallas_call(
        _ln_r1_body,
        grid=(T // bm,),
        in_specs=[
            pl.BlockSpec((bm, D), lambda i: (i, 0)),
            pl.BlockSpec((D, D), lambda i: (0, 0)),
            pl.BlockSpec((1, D), lambda i: (0, 0)),
            pl.BlockSpec((1, D), lambda i: (0, 0)),
        ],
        out_specs=pl.BlockSpec((bm, D), lambda i: (i, 0)),
        out_shape=jax.ShapeDtypeStruct((T, D), jnp.float32),
        interpret=interpret,
    )(xf, w1t, g, b)


def _mm_body(a_ref, b_ref, o_ref):
    o_ref[...] = jnp.dot(a_ref[...], b_ref[...],
                         preferred_element_type=jnp.float32)


def _mm(a, b, bm=512, bn=2048, interpret=False):
    M, Kd = a.shape
    N = b.shape[1]
    return pl.pallas_call(
        _mm_body,
        grid=(N // bn, M // bm),
        in_specs=[
            pl.BlockSpec((bm, Kd), lambda j, i: (i, 0)),
            pl.BlockSpec((Kd, bn), lambda j, i: (0, j)),
        ],
        out_specs=pl.BlockSpec((bm, bn), lambda j, i: (i, j)),
        out_shape=jax.ShapeDtypeStruct((M, N), jnp.float32),
        interpret=interpret,
    )(a, b)


def _topk_body(s_ref, xp_ref, idx_ref, base_ref):
    s = s_ref[...]
    xp = xp_ref[...]
    iota = lax.broadcasted_iota(jnp.int32, s.shape, 1)
    big = jnp.int32(2 ** 30)
    for j in range(K):
        m = jnp.max(s, axis=1, keepdims=True)
        eq = s == m
        ij = jnp.min(jnp.where(eq, iota, big), axis=1)
        sel = iota == ij[:, None]
        bj = jnp.sum(jnp.where(sel, xp, jnp.float32(0.0)), axis=1)
        s = jnp.where(sel, jnp.float32(-jnp.inf), s)
        idx_ref[j, :] = ij
        base_ref[j, :] = _gelu(bj)


def _topk(scores, xpat, bm=128, interpret=False):
    T, F = scores.shape
    return pl.pallas_call(
        _topk_body,
        grid=(T // bm,),
        in_specs=[
            pl.BlockSpec((bm, F), lambda i: (i, 0)),
            pl.BlockSpec((bm, F), lambda i: (i, 0)),
        ],
        out_specs=[
            pl.BlockSpec((K, bm), lambda i: (0, i)),
            pl.BlockSpec((K, bm), lambda i: (0, i)),
        ],
        out_shape=[
            jax.ShapeDtypeStruct((K, T), jnp.int32),
            jax.ShapeDtypeStruct((K, T), jnp.float32),
        ],
        interpret=interpret,
    )(scores, xpat)


_NW = 32
_VCH = 128
_DCH = 32
_NB = 2


@functools.lru_cache(maxsize=4)
def _make_sc_gather(b8):
    bpw = b8 // _NW

    @functools.partial(
        pl.kernel,
        out_type=[
            jax.ShapeDtypeStruct((b8, D_NEURON), jnp.float32),
            jax.ShapeDtypeStruct((b8, D_MODEL), jnp.float32),
        ],
        mesh=plsc.VectorSubcoreMesh(core_axis_name="c", subcore_axis_name="s"),
        scratch_types=[
            pltpu.VMEM((bpw,), jnp.int32),
            pltpu.VMEM((_VCH, D_NEURON), jnp.float32),
            pltpu.VMEM((_VCH, D_NEURON), jnp.float32),
        ] + [pltpu.VMEM((_DCH, D_MODEL), jnp.float32)] * _NB + [
            pltpu.SemaphoreType.DMA,
            pltpu.SemaphoreType.DMA,
        ] + [pltpu.SemaphoreType.DMA] * (2 * _NB),
    )
    def _sc_gather(vecs_hbm, dirs_hbm, idx_hbm, sems_out, dirsg_out,
                   idx_v, vbufA, vbufB, *drest):
        dbufs = list(drest[:_NB])
        gA, gB = drest[_NB], drest[_NB + 1]
        dg = list(drest[_NB + 2:2 * _NB + 2])
        ds = list(drest[2 * _NB + 2:])
        wid = lax.axis_index("s") * 2 + lax.axis_index("c")
        base = wid * bpw
        pltpu.sync_copy(idx_hbm.at[pl.ds(base, bpw)], idx_v)

        def ring(table, out_hbm, bufA, bufB, ch, nch):
            def gsrc(i):
                return table.at[idx_v.at[pl.ds(i * ch, ch)]]

            def orows(i):
                return out_hbm.at[pl.ds(base + i * ch, ch)]

            pltpu.async_copy(gsrc(0), bufA, gA)
            pltpu.async_copy(gsrc(1), bufB, gB)

            def body(p, c):
                e = 2 * p
                pltpu.make_async_copy(gsrc(e), bufA, gA).wait()
                pltpu.async_copy(bufA, orows(e), ds[0])
                pltpu.make_async_copy(bufA, orows(e), ds[0]).wait()
                pltpu.async_copy(gsrc(e + 2), bufA, gA)
                pltpu.make_async_copy(gsrc(e + 1), bufB, gB).wait()
                pltpu.async_copy(bufB, orows(e + 1), ds[1])
                pltpu.make_async_copy(bufB, orows(e + 1), ds[1]).wait()
                pltpu.async_copy(gsrc(e + 3), bufB, gB)
                return c

            lax.fori_loop(0, nch // 2 - 1, body, 0)
            e = nch - 2
            pltpu.make_async_copy(gsrc(e), bufA, gA).wait()
            pltpu.async_copy(bufA, orows(e), ds[0])
            pltpu.make_async_copy(gsrc(e + 1), bufB, gB).wait()
            pltpu.async_copy(bufB, orows(e + 1), ds[1])
            pltpu.make_async_copy(bufA, orows(e), ds[0]).wait()
            pltpu.make_async_copy(bufB, orows(e + 1), ds[1]).wait()

        ring(vecs_hbm, sems_out, vbufA, vbufB, _VCH, bpw // _VCH)
        ring(dirs_hbm, dirsg_out, dbufs[0], dbufs[1], _DCH, bpw // _DCH)

    return _sc_gather


def _df_body(sems_ref, base_ref, dirs_ref, inw_ref, inb_ref, outw_ref,
             outb_ref, w1_ref, b1_ref, w2_ref, b2_ref, rbd_ref, ebd_ref,
             o_ref):
    G = o_ref.shape[0]
    s = sems_ref[...].reshape(K * G, D_NEURON)
    qkv = jnp.dot(s, inw_ref[...], preferred_element_type=jnp.float32) + inb_ref[...]
    q = qkv[:, :D_NEURON]
    k = qkv[:, D_NEURON:2 * D_NEURON]
    v = qkv[:, 2 * D_NEURON:]
    rbd = rbd_ref[...]
    ebd = ebd_ref[...]
    ls = []
    for j in range(K):
        kj = k[j * G:(j + 1) * G, :]
        prod = jnp.concatenate(
            [q[i * G:(i + 1) * G, :] * kj for i in range(K)], axis=1)
        ls.append(jnp.dot(prod, rbd, preferred_element_type=jnp.float32))
    m = ls[0]
    for j in range(1, K):
        m = jnp.maximum(m, ls[j])
    es = [jnp.exp(l - m) for l in ls]
    ssum = es[0]
    for j in range(1, K):
        ssum = ssum + es[j]
    acc = None
    for j in range(K):
        pexp = jnp.dot(es[j] / ssum, ebd, preferred_element_type=jnp.float32)
        vj = v[j * G:(j + 1) * G, :]
        vt = jnp.concatenate([vj] * K, axis=1)
        t = pexp * vt
        acc = t if acc is None else acc + t
    att = jnp.dot(acc, outw_ref[...], preferred_element_type=jnp.float32) + outb_ref[...]
    hmid = _gelu(jnp.dot(att, w1_ref[...], preferred_element_type=jnp.float32) + b1_ref[...])
    wlog = jnp.dot(hmid, w2_ref[...], preferred_element_type=jnp.float32) + b2_ref[...]
    w = 1.0 / (1.0 + jnp.exp(-wlog))
    coef = base_ref[0].transpose() * w
    o = None
    for i in range(K):
        t = coef[:, i:i + 1] * dirs_ref[i]
        o = t if o is None else o + t
    o_ref[...] = o


def _df(sems3, baseT, dirs3, inw, inb, outw, outb, w1, b1, w2,
        b2, rbd, ebd, gtok=128, interpret=False):
    T = baseT.shape[1]
    full = lambda shape: pl.BlockSpec(shape, lambda i: tuple(0 for _ in shape))
    return pl.pallas_call(
        _df_body,
        grid=(T // gtok,),
        in_specs=[
            pl.BlockSpec((K, gtok, D_NEURON), lambda i: (0, i, 0)),
            pl.BlockSpec((1, K, gtok), lambda i: (0, 0, i)),
            pl.BlockSpec((K, gtok, D_MODEL), lambda i: (0, i, 0)),
            full((D_NEURON, 3 * D_NEURON)),
            full((1, 3 * D_NEURON)),
            full((K * D_NEURON, K * D_NEURON)),
            full((1, K * D_NEURON)),
            full((K * D_NEURON, K * D_NEURON // 2)),
            full((1, K * D_NEURON // 2)),
            full((K * D_NEURON // 2, K)),
            full((1, 1)),
            full((K * D_NEURON, K * N_HEADS)),
            full((K * N_HEADS, K * D_NEURON)),
        ],
        out_specs=pl.BlockSpec((gtok, D_MODEL), lambda i: (i, 0)),
        out_shape=jax.ShapeDtypeStruct((T, D_MODEL), jnp.float32),
        interpret=interpret,
    )(sems3, baseT.reshape(1, K, T), dirs3, inw, inb, outw, outb, w1, b1,
      w2, b2, rbd, ebd)


def kernel(x, top_k, neuron_patterns, neuron_vecs, neuron_out_dirs,
           W_router_1, W_router_2, ln_gamma, ln_beta, in_proj_w, in_proj_b,
           out_proj_w, out_proj_b, wp_w1, wp_b1, wp_w2, wp_b2):
    B, S, D = x.shape
    xf = x.reshape(-1, D)

    h = _ln_router1(xf, W_router_1.T, ln_gamma.reshape(1, -1),
                    ln_beta.reshape(1, -1))
    scores = _mm(h, W_router_2.T)
    xpat = _mm(xf, neuron_patterns.T)

    hmask = (np.arange(D_NEURON)[:, None] // HD ==
             np.arange(N_HEADS)[None, :]).astype(np.float32)
    rbd_np = np.kron(np.eye(K, dtype=np.float32),
                     hmask * np.float32(1.0 / np.sqrt(float(HD))))
    ebd_np = np.kron(np.eye(K, dtype=np.float32), hmask.T)
    rbd = jnp.asarray(rbd_np)
    ebd = jnp.asarray(ebd_np)

    eye8 = jnp.eye(K, dtype=jnp.float32)
    owbd = jnp.kron(eye8, out_proj_w.T)
    outb_t = jnp.tile(out_proj_b, K).reshape(1, -1)
    w1bd = jnp.kron(eye8, wp_w1.T)
    b1_t = jnp.tile(wp_b1, K).reshape(1, -1)
    w2bd = jnp.kron(eye8, wp_w2.reshape(-1, 1))

    T = xf.shape[0]
    idxT, baseT = _topk(scores, xpat)
    idx_flat = idxT.reshape(-1)
    sems_flat, dirs_flat = _make_sc_gather(T * K)(neuron_vecs,
                                                  neuron_out_dirs, idx_flat)
    sems3 = sems_flat.reshape(K, T, D_NEURON)
    dirs3 = dirs_flat.reshape(K, T, D_MODEL)
    out = _df(sems3, baseT, dirs3,
              in_proj_w.T, in_proj_b.reshape(1, -1),
              owbd, outb_t, w1bd, b1_t, w2bd,
              wp_b2.reshape(1, 1), rbd, ebd)
    return out.reshape(B, S, D)


# speedup vs baseline: 1.0569x; 1.0235x over previous
import functools
import numpy as np
import jax
import jax.numpy as jnp
from jax import lax
from jax.experimental import pallas as pl
from jax.experimental.pallas import tpu as pltpu
from jax.experimental.pallas import tpu_sc as plsc

D_MODEL = 1024
D_FF = 4096
D_NEURON = 128
N_HEADS = 8
HD = D_NEURON // N_HEADS
K = 8

def _gelu(v):
    return v * (lax.erf(v / np.sqrt(2.0)) + 1.0) / 2.0


def _ln_r1_body(x_ref, w1t_ref, g_ref, b_ref, h_ref):
    x = x_ref[...]
    m = jnp.mean(x, axis=1, keepdims=True)
    xc = x - m
    v = jnp.mean(xc * xc, axis=1, keepdims=True)
    xn = xc / jnp.sqrt(v + 1e-5) * g_ref[...] + b_ref[...]
    h_ref[...] = _gelu(jnp.dot(xn, w1t_ref[...],
                               preferred_element_type=jnp.float32))


def _ln_router1(xf, w1t, g, b, bm=512, interpret=False):
    T, D = xf.shape
    return pl.pallas_call(
        _ln_r1_body,
        grid=(T // bm,),
        in_specs=[
            pl.BlockSpec((bm, D), lambda i: (i, 0)),
            pl.BlockSpec((D, D), lambda i: (0, 0)),
            pl.BlockSpec((1, D), lambda i: (0, 0)),
            pl.BlockSpec((1, D), lambda i: (0, 0)),
        ],
        out_specs=pl.BlockSpec((bm, D), lambda i: (i, 0)),
        out_shape=jax.ShapeDtypeStruct((T, D), jnp.float32),
        interpret=interpret,
    )(xf, w1t, g, b)


def _mm_body(a_ref, b_ref, o_ref):
    o_ref[...] = jnp.dot(a_ref[...], b_ref[...],
                         preferred_element_type=jnp.float32)


def _mm(a, b, bm=512, bn=2048, interpret=False):
    M, Kd = a.shape
    N = b.shape[1]
    return pl.pallas_call(
        _mm_body,
        grid=(N // bn, M // bm),
        in_specs=[
            pl.BlockSpec((bm, Kd), lambda j, i: (i, 0)),
            pl.BlockSpec((Kd, bn), lambda j, i: (0, j)),
        ],
        out_specs=pl.BlockSpec((bm, bn), lambda j, i: (i, j)),
        out_shape=jax.ShapeDtypeStruct((M, N), jnp.float32),
        interpret=interpret,
    )(a, b)


def _topk_body(s_ref, xp_ref, idx_ref, base_ref):
    s = s_ref[...]
    xp = xp_ref[...]
    iota = lax.broadcasted_iota(jnp.int32, s.shape, 1)
    big = jnp.int32(2 ** 30)
    for j in range(K):
        m = jnp.max(s, axis=1, keepdims=True)
        eq = s == m
        ij = jnp.min(jnp.where(eq, iota, big), axis=1)
        sel = iota == ij[:, None]
        bj = jnp.sum(jnp.where(sel, xp, jnp.float32(0.0)), axis=1)
        s = jnp.where(sel, jnp.float32(-jnp.inf), s)
        idx_ref[j, :] = ij
        base_ref[j, :] = _gelu(bj)


def _topk(scores, xpat, bm=128, interpret=False):
    T, F = scores.shape
    return pl.pallas_call(
        _topk_body,
        grid=(T // bm,),
        in_specs=[
            pl.BlockSpec((bm, F), lambda i: (i, 0)),
            pl.BlockSpec((bm, F), lambda i: (i, 0)),
        ],
        out_specs=[
            pl.BlockSpec((K, bm), lambda i: (0, i)),
            pl.BlockSpec((K, bm), lambda i: (0, i)),
        ],
        out_shape=[
            jax.ShapeDtypeStruct((K, T), jnp.int32),
            jax.ShapeDtypeStruct((K, T), jnp.float32),
        ],
        interpret=interpret,
    )(scores, xpat)


_NW = 32
_VCH = 128
_DCH = 32
_NB = 2


@functools.lru_cache(maxsize=4)
def _make_sc_gather(b8):
    bpw = b8 // _NW

    @functools.partial(
        pl.kernel,
        out_type=[
            jax.ShapeDtypeStruct((b8, D_NEURON), jnp.float32),
            jax.ShapeDtypeStruct((b8, D_MODEL), jnp.float32),
        ],
        mesh=plsc.VectorSubcoreMesh(core_axis_name="c", subcore_axis_name="s"),
        scratch_types=[
            pltpu.VMEM((bpw,), jnp.int32),
            pltpu.VMEM((_VCH, D_NEURON), jnp.float32),
            pltpu.VMEM((_VCH, D_NEURON), jnp.float32),
        ] + [pltpu.VMEM((_DCH, D_MODEL), jnp.float32)] * _NB + [
            pltpu.SemaphoreType.DMA,
            pltpu.SemaphoreType.DMA,
        ] + [pltpu.SemaphoreType.DMA] * (2 * _NB),
    )
    def _sc_gather(vecs_hbm, dirs_hbm, idx_hbm, sems_out, dirsg_out,
                   idx_v, vbufA, vbufB, *drest):
        dbufs = list(drest[:_NB])
        gA, gB = drest[_NB], drest[_NB + 1]
        dg = list(drest[_NB + 2:2 * _NB + 2])
        ds = list(drest[2 * _NB + 2:])
        wid = lax.axis_index("s") * 2 + lax.axis_index("c")
        base = wid * bpw
        pltpu.sync_copy(idx_hbm.at[pl.ds(base, bpw)], idx_v)

        def ring(table, out_hbm, bufA, bufB, ch, nch):
            def gsrc(i):
                return table.at[idx_v.at[pl.ds(i * ch, ch)]]

            def orows(i):
                return out_hbm.at[pl.ds(base + i * ch, ch)]

            pltpu.async_copy(gsrc(0), bufA, gA)
            pltpu.async_copy(gsrc(1), bufB, gB)

            def body(p, c):
                e = 2 * p
                pltpu.make_async_copy(gsrc(e), bufA, gA).wait()
                pltpu.async_copy(bufA, orows(e), ds[0])
                pltpu.make_async_copy(bufA, orows(e), ds[0]).wait()
                pltpu.async_copy(gsrc(e + 2), bufA, gA)
                pltpu.make_async_copy(gsrc(e + 1), bufB, gB).wait()
                pltpu.async_copy(bufB, orows(e + 1), ds[1])
                pltpu.make_async_copy(bufB, orows(e + 1), ds[1]).wait()
                pltpu.async_copy(gsrc(e + 3), bufB, gB)
                return c

            lax.fori_loop(0, nch // 2 - 1, body, 0)
            e = nch - 2
            pltpu.make_async_copy(gsrc(e), bufA, gA).wait()
            pltpu.async_copy(bufA, orows(e), ds[0])
            pltpu.make_async_copy(gsrc(e + 1), bufB, gB).wait()
            pltpu.async_copy(bufB, orows(e + 1), ds[1])
            pltpu.make_async_copy(bufA, orows(e), ds[0]).wait()
            pltpu.make_async_copy(bufB, orows(e + 1), ds[1]).wait()

        ring(vecs_hbm, sems_out, vbufA, vbufB, _VCH, bpw // _VCH)
        ring(dirs_hbm, dirsg_out, dbufs[0], dbufs[1], _DCH, bpw // _DCH)

    return _sc_gather


def _df_body(sems_ref, base_ref, dirs_ref, inw_ref, inb_ref, outw_ref,
             outb_ref, w1_ref, b1_ref, w2_ref, b2_ref, rbd_ref, ebd_ref,
             o_ref):
    G = o_ref.shape[0]
    s = sems_ref[...].reshape(K * G, D_NEURON)
    qkv = jnp.dot(s, inw_ref[...], preferred_element_type=jnp.float32) + inb_ref[...]
    q = qkv[:, :D_NEURON]
    k = qkv[:, D_NEURON:2 * D_NEURON]
    v = qkv[:, 2 * D_NEURON:]
    rbd = rbd_ref[...]
    ebd = ebd_ref[...]
    ls = []
    for j in range(K):
        kj = k[j * G:(j + 1) * G, :]
        prod = jnp.concatenate(
            [q[i * G:(i + 1) * G, :] * kj for i in range(K)], axis=1)
        ls.append(jnp.dot(prod, rbd, preferred_element_type=jnp.float32))
    m = ls[0]
    for j in range(1, K):
        m = jnp.maximum(m, ls[j])
    es = [jnp.exp(l - m) for l in ls]
    ssum = es[0]
    for j in range(1, K):
        ssum = ssum + es[j]
    acc = None
    for j in range(K):
        pexp = jnp.dot(es[j] / ssum, ebd, preferred_element_type=jnp.float32)
        vj = v[j * G:(j + 1) * G, :]
        vt = jnp.concatenate([vj] * K, axis=1)
        t = pexp * vt
        acc = t if acc is None else acc + t
    att = jnp.dot(acc, outw_ref[...], preferred_element_type=jnp.float32) + outb_ref[...]
    hmid = _gelu(jnp.dot(att, w1_ref[...], preferred_element_type=jnp.float32) + b1_ref[...])
    wlog = jnp.dot(hmid, w2_ref[...], preferred_element_type=jnp.float32) + b2_ref[...]
    w = 1.0 / (1.0 + jnp.exp(-wlog))
    coef = base_ref[0].transpose() * w
    o = None
    for i in range(K):
        t = coef[:, i:i + 1] * dirs_ref[i]
        o = t if o is None else o + t
    o_ref[...] = o


def _df(sems3, baseT, dirs3, inw, inb, outw, outb, w1, b1, w2,
        b2, rbd, ebd, gtok=256, interpret=False):
    T = baseT.shape[1]
    full = lambda shape: pl.BlockSpec(shape, lambda i: tuple(0 for _ in shape))
    return pl.pallas_call(
        _df_body,
        grid=(T // gtok,),
        in_specs=[
            pl.BlockSpec((K, gtok, D_NEURON), lambda i: (0, i, 0)),
            pl.BlockSpec((1, K, gtok), lambda i: (0, 0, i)),
            pl.BlockSpec((K, gtok, D_MODEL), lambda i: (0, i, 0)),
            full((D_NEURON, 3 * D_NEURON)),
            full((1, 3 * D_NEURON)),
            full((K * D_NEURON, K * D_NEURON)),
            full((1, K * D_NEURON)),
            full((K * D_NEURON, K * D_NEURON // 2)),
            full((1, K * D_NEURON // 2)),
            full((K * D_NEURON // 2, K)),
            full((1, 1)),
            full((K * D_NEURON, K * N_HEADS)),
            full((K * N_HEADS, K * D_NEURON)),
        ],
        out_specs=pl.BlockSpec((gtok, D_MODEL), lambda i: (i, 0)),
        out_shape=jax.ShapeDtypeStruct((T, D_MODEL), jnp.float32),
        interpret=interpret,
    )(sems3, baseT.reshape(1, K, T), dirs3, inw, inb, outw, outb, w1, b1,
      w2, b2, rbd, ebd)


def kernel(x, top_k, neuron_patterns, neuron_vecs, neuron_out_dirs,
           W_router_1, W_router_2, ln_gamma, ln_beta, in_proj_w, in_proj_b,
           out_proj_w, out_proj_b, wp_w1, wp_b1, wp_w2, wp_b2):
    B, S, D = x.shape
    xf = x.reshape(-1, D)

    h = _ln_router1(xf, W_router_1.T, ln_gamma.reshape(1, -1),
                    ln_beta.reshape(1, -1))
    scores = _mm(h, W_router_2.T)
    xpat = _mm(xf, neuron_patterns.T)

    hmask = (np.arange(D_NEURON)[:, None] // HD ==
             np.arange(N_HEADS)[None, :]).astype(np.float32)
    rbd_np = np.kron(np.eye(K, dtype=np.float32),
                     hmask * np.float32(1.0 / np.sqrt(float(HD))))
    ebd_np = np.kron(np.eye(K, dtype=np.float32), hmask.T)
    rbd = jnp.asarray(rbd_np)
    ebd = jnp.asarray(ebd_np)

    eye8 = jnp.eye(K, dtype=jnp.float32)
    owbd = jnp.kron(eye8, out_proj_w.T)
    outb_t = jnp.tile(out_proj_b, K).reshape(1, -1)
    w1bd = jnp.kron(eye8, wp_w1.T)
    b1_t = jnp.tile(wp_b1, K).reshape(1, -1)
    w2bd = jnp.kron(eye8, wp_w2.reshape(-1, 1))

    T = xf.shape[0]
    idxT, baseT = _topk(scores, xpat)
    idx_flat = idxT.reshape(-1)
    sems_flat, dirs_flat = _make_sc_gather(T * K)(neuron_vecs,
                                                  neuron_out_dirs, idx_flat)
    sems3 = sems_flat.reshape(K, T, D_NEURON)
    dirs3 = dirs_flat.reshape(K, T, D_MODEL)
    out = _df(sems3, baseT, dirs3,
              in_proj_w.T, in_proj_b.reshape(1, -1),
              owbd, outb_t, w1bd, b1_t, w2bd,
              wp_b2.reshape(1, 1), rbd, ebd)
    return out.reshape(B, S, D)


# speedup vs baseline: 1.0591x; 1.0021x over previous
import functools
import numpy as np
import jax
import jax.numpy as jnp
from jax import lax
from jax.experimental import pallas as pl
from jax.experimental.pallas import tpu as pltpu
from jax.experimental.pallas import tpu_sc as plsc

D_MODEL = 1024
D_FF = 4096
D_NEURON = 128
N_HEADS = 8
HD = D_NEURON // N_HEADS
K = 8

def _gelu(v):
    return v * (lax.erf(v / np.sqrt(2.0)) + 1.0) / 2.0


def _ln_r1_body(x_ref, w1t_ref, g_ref, b_ref, h_ref):
    x = x_ref[...]
    m = jnp.mean(x, axis=1, keepdims=True)
    xc = x - m
    v = jnp.mean(xc * xc, axis=1, keepdims=True)
    xn = xc / jnp.sqrt(v + 1e-5) * g_ref[...] + b_ref[...]
    h_ref[...] = _gelu(jnp.dot(xn, w1t_ref[...],
                               preferred_element_type=jnp.float32))


def _ln_router1(xf, w1t, g, b, bm=512, interpret=False):
    T, D = xf.shape
    return pl.pallas_call(
        _ln_r1_body,
        grid=(T // bm,),
        in_specs=[
            pl.BlockSpec((bm, D), lambda i: (i, 0)),
            pl.BlockSpec((D, D), lambda i: (0, 0)),
            pl.BlockSpec((1, D), lambda i: (0, 0)),
            pl.BlockSpec((1, D), lambda i: (0, 0)),
        ],
        out_specs=pl.BlockSpec((bm, D), lambda i: (i, 0)),
        out_shape=jax.ShapeDtypeStruct((T, D), jnp.float32),
        interpret=interpret,
    )(xf, w1t, g, b)


def _mm_body(a_ref, b_ref, o_ref):
    o_ref[...] = jnp.dot(a_ref[...], b_ref[...],
                         preferred_element_type=jnp.float32)


def _mm(a, b, bm=512, bn=2048, interpret=False):
    M, Kd = a.shape
    N = b.shape[1]
    return pl.pallas_call(
        _mm_body,
        grid=(N // bn, M // bm),
        in_specs=[
            pl.BlockSpec((bm, Kd), lambda j, i: (i, 0)),
            pl.BlockSpec((Kd, bn), lambda j, i: (0, j)),
        ],
        out_specs=pl.BlockSpec((bm, bn), lambda j, i: (i, j)),
        out_shape=jax.ShapeDtypeStruct((M, N), jnp.float32),
        interpret=interpret,
    )(a, b)


def _topk_body(s_ref, xp_ref, idx_ref, base_ref):
    s = s_ref[...]
    xp = xp_ref[...]
    iota = lax.broadcasted_iota(jnp.int32, s.shape, 1)
    big = jnp.int32(2 ** 30)
    for j in range(K):
        m = jnp.max(s, axis=1, keepdims=True)
        eq = s == m
        ij = jnp.min(jnp.where(eq, iota, big), axis=1)
        sel = iota == ij[:, None]
        bj = jnp.sum(jnp.where(sel, xp, jnp.float32(0.0)), axis=1)
        s = jnp.where(sel, jnp.float32(-jnp.inf), s)
        idx_ref[j, :] = ij
        base_ref[j, :] = _gelu(bj)


def _topk(scores, xpat, bm=128, interpret=False):
    T, F = scores.shape
    return pl.pallas_call(
        _topk_body,
        grid=(T // bm,),
        in_specs=[
            pl.BlockSpec((bm, F), lambda i: (i, 0)),
            pl.BlockSpec((bm, F), lambda i: (i, 0)),
        ],
        out_specs=[
            pl.BlockSpec((K, bm), lambda i: (0, i)),
            pl.BlockSpec((K, bm), lambda i: (0, i)),
        ],
        out_shape=[
            jax.ShapeDtypeStruct((K, T), jnp.int32),
            jax.ShapeDtypeStruct((K, T), jnp.float32),
        ],
        interpret=interpret,
    )(scores, xpat)


_NW = 32
_VCH = 128
_DCH = 32
_NB = 2


@functools.lru_cache(maxsize=4)
def _make_sc_gather(b8):
    bpw = b8 // _NW

    @functools.partial(
        pl.kernel,
        out_type=[
            jax.ShapeDtypeStruct((b8, D_NEURON), jnp.float32),
            jax.ShapeDtypeStruct((b8, D_MODEL), jnp.float32),
        ],
        mesh=plsc.VectorSubcoreMesh(core_axis_name="c", subcore_axis_name="s"),
        scratch_types=[
            pltpu.VMEM((bpw,), jnp.int32),
            pltpu.VMEM((_VCH, D_NEURON), jnp.float32),
            pltpu.VMEM((_VCH, D_NEURON), jnp.float32),
        ] + [pltpu.VMEM((_DCH, D_MODEL), jnp.float32)] * _NB + [
            pltpu.SemaphoreType.DMA,
            pltpu.SemaphoreType.DMA,
        ] + [pltpu.SemaphoreType.DMA] * (2 * _NB),
    )
    def _sc_gather(vecs_hbm, dirs_hbm, idx_hbm, sems_out, dirsg_out,
                   idx_v, vbufA, vbufB, *drest):
        dbufs = list(drest[:_NB])
        gA, gB = drest[_NB], drest[_NB + 1]
        dg = list(drest[_NB + 2:2 * _NB + 2])
        ds = list(drest[2 * _NB + 2:])
        wid = lax.axis_index("s") * 2 + lax.axis_index("c")
        base = wid * bpw
        pltpu.sync_copy(idx_hbm.at[pl.ds(base, bpw)], idx_v)

        def ring(table, out_hbm, bufA, bufB, ch, nch):
            def gsrc(i):
                return table.at[idx_v.at[pl.ds(i * ch, ch)]]

            def orows(i):
                return out_hbm.at[pl.ds(base + i * ch, ch)]

            pltpu.async_copy(gsrc(0), bufA, gA)
            pltpu.async_copy(gsrc(1), bufB, gB)

            def body(p, c):
                e = 2 * p
                pltpu.make_async_copy(gsrc(e), bufA, gA).wait()
                pltpu.async_copy(bufA, orows(e), ds[0])
                pltpu.make_async_copy(bufA, orows(e), ds[0]).wait()
                pltpu.async_copy(gsrc(e + 2), bufA, gA)
                pltpu.make_async_copy(gsrc(e + 1), bufB, gB).wait()
                pltpu.async_copy(bufB, orows(e + 1), ds[1])
                pltpu.make_async_copy(bufB, orows(e + 1), ds[1]).wait()
                pltpu.async_copy(gsrc(e + 3), bufB, gB)
                return c

            lax.fori_loop(0, nch // 2 - 1, body, 0)
            e = nch - 2
            pltpu.make_async_copy(gsrc(e), bufA, gA).wait()
            pltpu.async_copy(bufA, orows(e), ds[0])
            pltpu.make_async_copy(gsrc(e + 1), bufB, gB).wait()
            pltpu.async_copy(bufB, orows(e + 1), ds[1])
            pltpu.make_async_copy(bufA, orows(e), ds[0]).wait()
            pltpu.make_async_copy(bufB, orows(e + 1), ds[1]).wait()

        ring(vecs_hbm, sems_out, vbufA, vbufB, _VCH, bpw // _VCH)
        ring(dirs_hbm, dirsg_out, dbufs[0], dbufs[1], _DCH, bpw // _DCH)

    return _sc_gather


def _df_body(sems_ref, base_ref, dirs_ref, inw_ref, inb_ref, outw_ref,
             outb_ref, w1_ref, b1_ref, w2_ref, b2_ref, rbd_ref, ebd_ref,
             o_ref):
    G = o_ref.shape[0]
    s = sems_ref[...].reshape(K * G, D_NEURON)
    qkv = jnp.dot(s, inw_ref[...], preferred_element_type=jnp.float32) + inb_ref[...]
    q = qkv[:, :D_NEURON]
    k = qkv[:, D_NEURON:2 * D_NEURON]
    v = qkv[:, 2 * D_NEURON:]
    rbd = rbd_ref[...]
    ebd = ebd_ref[...]
    ls = []
    for j in range(K):
        kj = k[j * G:(j + 1) * G, :]
        prod = jnp.concatenate(
            [q[i * G:(i + 1) * G, :] * kj for i in range(K)], axis=1)
        ls.append(jnp.dot(prod, rbd, preferred_element_type=jnp.float32))
    m = ls[0]
    for j in range(1, K):
        m = jnp.maximum(m, ls[j])
    es = [jnp.exp(l - m) for l in ls]
    ssum = es[0]
    for j in range(1, K):
        ssum = ssum + es[j]
    acc = None
    for j in range(K):
        pexp = jnp.dot(es[j] / ssum, ebd, preferred_element_type=jnp.float32)
        vj = v[j * G:(j + 1) * G, :]
        vt = jnp.concatenate([vj] * K, axis=1)
        t = pexp * vt
        acc = t if acc is None else acc + t
    att = jnp.dot(acc, outw_ref[...], preferred_element_type=jnp.float32) + outb_ref[...]
    hmid = _gelu(jnp.dot(att, w1_ref[...], preferred_element_type=jnp.float32) + b1_ref[...])
    wlog = jnp.dot(hmid, w2_ref[...], preferred_element_type=jnp.float32) + b2_ref[...]
    w = 1.0 / (1.0 + jnp.exp(-wlog))
    coef = base_ref[0].transpose() * w
    o = None
    for i in range(K):
        t = coef[:, i:i + 1] * dirs_ref[i]
        o = t if o is None else o + t
    o_ref[...] = o


def _df(sems3, baseT, dirs3, inw, inb, outw, outb, w1, b1, w2,
        b2, rbd, ebd, gtok=512, interpret=False):
    T = baseT.shape[1]
    full = lambda shape: pl.BlockSpec(shape, lambda i: tuple(0 for _ in shape))
    return pl.pallas_call(
        _df_body,
        grid=(T // gtok,),
        in_specs=[
            pl.BlockSpec((K, gtok, D_NEURON), lambda i: (0, i, 0)),
            pl.BlockSpec((1, K, gtok), lambda i: (0, 0, i)),
            pl.BlockSpec((K, gtok, D_MODEL), lambda i: (0, i, 0)),
            full((D_NEURON, 3 * D_NEURON)),
            full((1, 3 * D_NEURON)),
            full((K * D_NEURON, K * D_NEURON)),
            full((1, K * D_NEURON)),
            full((K * D_NEURON, K * D_NEURON // 2)),
            full((1, K * D_NEURON // 2)),
            full((K * D_NEURON // 2, K)),
            full((1, 1)),
            full((K * D_NEURON, K * N_HEADS)),
            full((K * N_HEADS, K * D_NEURON)),
        ],
        out_specs=pl.BlockSpec((gtok, D_MODEL), lambda i: (i, 0)),
        out_shape=jax.ShapeDtypeStruct((T, D_MODEL), jnp.float32),
        interpret=interpret,
    )(sems3, baseT.reshape(1, K, T), dirs3, inw, inb, outw, outb, w1, b1,
      w2, b2, rbd, ebd)


def kernel(x, top_k, neuron_patterns, neuron_vecs, neuron_out_dirs,
           W_router_1, W_router_2, ln_gamma, ln_beta, in_proj_w, in_proj_b,
           out_proj_w, out_proj_b, wp_w1, wp_b1, wp_w2, wp_b2):
    B, S, D = x.shape
    xf = x.reshape(-1, D)

    h = _ln_router1(xf, W_router_1.T, ln_gamma.reshape(1, -1),
                    ln_beta.reshape(1, -1))
    scores = _mm(h, W_router_2.T)
    xpat = _mm(xf, neuron_patterns.T)

    hmask = (np.arange(D_NEURON)[:, None] // HD ==
             np.arange(N_HEADS)[None, :]).astype(np.float32)
    rbd_np = np.kron(np.eye(K, dtype=np.float32),
                     hmask * np.float32(1.0 / np.sqrt(float(HD))))
    ebd_np = np.kron(np.eye(K, dtype=np.float32), hmask.T)
    rbd = jnp.asarray(rbd_np)
    ebd = jnp.asarray(ebd_np)

    eye8 = jnp.eye(K, dtype=jnp.float32)
    owbd = jnp.kron(eye8, out_proj_w.T)
    outb_t = jnp.tile(out_proj_b, K).reshape(1, -1)
    w1bd = jnp.kron(eye8, wp_w1.T)
    b1_t = jnp.tile(wp_b1, K).reshape(1, -1)
    w2bd = jnp.kron(eye8, wp_w2.reshape(-1, 1))

    T = xf.shape[0]
    idxT, baseT = _topk(scores, xpat)
    idx_flat = idxT.reshape(-1)
    sems_flat, dirs_flat = _make_sc_gather(T * K)(neuron_vecs,
                                                  neuron_out_dirs, idx_flat)
    sems3 = sems_flat.reshape(K, T, D_NEURON)
    dirs3 = dirs_flat.reshape(K, T, D_MODEL)
    out = _df(sems3, baseT, dirs3,
              in_proj_w.T, in_proj_b.reshape(1, -1),
              owbd, outb_t, w1bd, b1_t, w2bd,
              wp_b2.reshape(1, 1), rbd, ebd)
    return out.reshape(B, S, D)
